# Initial kernel scaffold; baseline (speedup 1.0000x reference)
import jax
import jax.numpy as jnp
from jax.experimental import pallas as pl


def kernel(input_id_sequence):
    raise NotImplementedError("write your pallas kernel here")


# speedup vs baseline: 1.7368x; 1.7368x over previous
import functools

import jax
import jax.numpy as jnp
from jax import lax
from jax.experimental import pallas as pl
from jax.experimental.pallas import tpu as pltpu
from jax.experimental.pallas import tpu_sc as plsc

_B = 4
_S = 8190
_FLAT = _B * _S
_NW = 32
_CH = 1024
_HALO = 8
_BUF = _CH + _HALO
_LAST_BASE = _FLAT - _BUF
_TAIL = _FLAT - (_NW - 1) * _CH
_NVEC = _CH // 16
_DOFF = 8
_BUFSZ = _DOFF + 16 + _CH


def _router_body(x_hbm, pos_hbm, beh_hbm, buf, pos_buf, beh_buf):
    nc = 2
    wid = lax.axis_index("s") * nc + lax.axis_index("c")
    s = pl.multiple_of(wid * _CH, _HALO)
    base = pl.multiple_of(
        jnp.minimum(jnp.maximum(s - _HALO, 0), _LAST_BASE), _HALO)
    off = s - base
    jstart = lax.rem(s, _S)

    pltpu.sync_copy(x_hbm.at[pl.ds(base, _BUF)], buf.at[pl.ds(_DOFF, _BUF)])

    lane = lax.iota(jnp.int32, 16)

    def body(i, _):
        k = i * 16
        li0 = _DOFF + off + k
        x = buf[pl.ds(li0, 16)]
        x1 = buf[pl.ds(li0 - 1, 16)]
        x2 = buf[pl.ds(li0 - 2, 16)]
        x3 = buf[pl.ds(li0 - 3, 16)]
        j_un = jstart + k + lane
        j = jnp.where(j_un >= _S, j_un - _S, j_un)
        d = (j + 3) & 3
        v = jnp.where(d == 1, x1, jnp.where(d == 2, x2, x3))
        zero = jnp.zeros((16,), jnp.int32)
        pos_kill = (j == 0) | (j == (_S - 1)) | (x == 0) | (x == 1)
        pos = jnp.where(pos_kill, zero, d + 1)
        beh_kill = (j == 0) | (d == 0) | (v == 1)
        beh = jnp.where(beh_kill, zero, v)
        pos_buf[pl.ds(k, 16)] = pos
        beh_buf[pl.ds(k, 16)] = beh
        return 0

    lax.fori_loop(0, _NVEC, body, 0)

    @pl.when(wid < _NW - 1)
    def _():
        pltpu.sync_copy(pos_buf, pos_hbm.at[pl.ds(s, _CH)])
        pltpu.sync_copy(beh_buf, beh_hbm.at[pl.ds(s, _CH)])

    @pl.when(wid == _NW - 1)
    def _():
        pltpu.sync_copy(pos_buf.at[pl.ds(0, _TAIL)], pos_hbm.at[pl.ds(s, _TAIL)])
        pltpu.sync_copy(beh_buf.at[pl.ds(0, _TAIL)], beh_hbm.at[pl.ds(s, _TAIL)])


@jax.jit
def kernel(input_id_sequence):
    xf = input_id_sequence.reshape(_FLAT)
    mesh = plsc.VectorSubcoreMesh(core_axis_name="c", subcore_axis_name="s")
    run = pl.kernel(
        _router_body,
        mesh=mesh,
        out_type=(
            jax.ShapeDtypeStruct((_FLAT,), jnp.int32),
            jax.ShapeDtypeStruct((_FLAT,), jnp.int32),
        ),
        scratch_types=[
            pltpu.VMEM((_BUFSZ,), jnp.int32),
            pltpu.VMEM((_CH,), jnp.int32),
            pltpu.VMEM((_CH,), jnp.int32),
        ],
    )
    pos_flat, beh_flat = run(xf)
    return (pos_flat.reshape(_B, _S), beh_flat.reshape(_B, _S))


# speedup vs baseline: 1.8101x; 1.0422x over previous
import jax
import jax.numpy as jnp
from jax import lax
from jax.experimental import pallas as pl
from jax.experimental.pallas import tpu as pltpu
from jax.experimental.pallas import tpu_sc as plsc

_B = 4
_S = 8190
_NW = 16
_CPR = 4
_CH = 2048
_LCH = _S - (_CPR - 1) * _CH
_HALO = 8
_BUF = _CH + _HALO
_LBUF = _CH + _HALO
_DOFF = 8
_BUFSZ = _DOFF + _HALO + _CH
_NVEC = _CH // 16


def _router_body(x_hbm, pos_hbm, beh_hbm, buf, pos_buf, beh_buf):
    wid = lax.axis_index("s")
    b = wid // _CPR
    c = wid % _CPR
    cs = pl.multiple_of(c * _CH, _HALO)
    base = pl.multiple_of(jnp.maximum(cs - _HALO, 0), _HALO)
    off = cs - base

    pltpu.sync_copy(x_hbm.at[b, pl.ds(base, _BUF)],
                    buf.at[pl.ds(_DOFF, _BUF)])

    lane = lax.iota(jnp.int32, 16)
    d = (lane + 3) & 3
    is1 = d == 1
    is2 = d == 2
    d0 = d == 0
    dp1 = d + 1
    zero = jnp.zeros((16,), jnp.int32)

    def body(i, _):
        k = i * 16
        li0 = _DOFF + off + k
        x = buf[pl.ds(li0, 16)]
        x1 = buf[pl.ds(li0 - 1, 16)]
        x2 = buf[pl.ds(li0 - 2, 16)]
        x3 = buf[pl.ds(li0 - 3, 16)]
        j = cs + k + lane
        v = jnp.where(is1, x1, jnp.where(is2, x2, x3))
        edge = (j == 0) | (j == (_S - 1))
        pos_kill = edge | (x == 0) | (x == 1)
        pos = jnp.where(pos_kill, zero, dp1)
        beh_kill = (j == 0) | d0 | (v == 1)
        beh = jnp.where(beh_kill, zero, v)
        pos_buf[pl.ds(k, 16)] = pos
        beh_buf[pl.ds(k, 16)] = beh
        return 0

    lax.fori_loop(0, _NVEC, body, 0)

    pltpu.sync_copy(pos_buf, pos_hbm.at[b, pl.ds(cs, _CH)])
    pltpu.sync_copy(beh_buf, beh_hbm.at[b, pl.ds(cs, _CH)])


@jax.jit
def kernel(input_id_sequence):
    mesh = plsc.VectorSubcoreMesh(
        core_axis_name="c", subcore_axis_name="s", num_cores=1)
    run = pl.kernel(
        _router_body,
        mesh=mesh,
        out_type=(
            jax.ShapeDtypeStruct((_B, _S), jnp.int32),
            jax.ShapeDtypeStruct((_B, _S), jnp.int32),
        ),
        scratch_types=[
            pltpu.VMEM((_BUFSZ,), jnp.int32),
            pltpu.VMEM((_CH,), jnp.int32),
            pltpu.VMEM((_CH,), jnp.int32),
        ],
        compiler_params=pltpu.CompilerParams(use_tc_tiling_on_sc=False),
    )
    return run(input_id_sequence)


# speedup vs baseline: 1.8137x; 1.0020x over previous
import jax
import jax.numpy as jnp
from jax import lax
from jax.experimental import pallas as pl
from jax.experimental.pallas import tpu as pltpu
from jax.experimental.pallas import tpu_sc as plsc

_B = 4
_S = 8190
_CPR = 4
_CH = 2048
_HALO = 8
_BUF = _CH + _HALO
_DOFF = 8
_BUFSZ = _DOFF + _HALO + _CH
_NVEC = _CH // 16
_UNROLL = 4


def _router_body(x_hbm, pos_hbm, beh_hbm, buf, pos_buf, beh_buf, sem1, sem2):
    wid = lax.axis_index("s")
    b = wid // _CPR
    c = wid % _CPR
    cs = pl.multiple_of(c * _CH, _HALO)
    base = pl.multiple_of(jnp.maximum(cs - _HALO, 0), _HALO)
    off = cs - base

    pltpu.sync_copy(x_hbm.at[b, pl.ds(base, _BUF)],
                    buf.at[pl.ds(_DOFF, _BUF)])

    lane = lax.iota(jnp.int32, 16)
    d = (lane + 3) & 3
    is1 = d == 1
    is2 = d == 2
    d0 = d == 0
    dp1 = d + 1
    zero = jnp.zeros((16,), jnp.int32)
    one_u = jnp.ones((16,), jnp.uint32)

    def do_vec(k):
        li0 = _DOFF + off + k
        x = buf[pl.ds(li0, 16)]
        x1 = buf[pl.ds(li0 - 1, 16)]
        x2 = buf[pl.ds(li0 - 2, 16)]
        x3 = buf[pl.ds(li0 - 3, 16)]
        v = jnp.where(is1, x1, jnp.where(is2, x2, x3))
        pos_kill = plsc.bitcast(x, jnp.uint32) <= one_u
        pos = jnp.where(pos_kill, zero, dp1)
        beh_kill = d0 | (v == 1)
        beh = jnp.where(beh_kill, zero, v)
        pos_buf[pl.ds(k, 16)] = pos
        beh_buf[pl.ds(k, 16)] = beh

    def body(i, _):
        k0 = i * (16 * _UNROLL)
        for u in range(_UNROLL):
            do_vec(k0 + u * 16)
        return 0

    lax.fori_loop(0, _NVEC // _UNROLL, body, 0)

    @pl.when(c == 0)
    def _():
        l0 = lane == 0
        pos_buf[pl.ds(0, 16)] = jnp.where(l0, zero, pos_buf[pl.ds(0, 16)])
        beh_buf[pl.ds(0, 16)] = jnp.where(l0, zero, beh_buf[pl.ds(0, 16)])

    @pl.when(c == _CPR - 1)
    def _():
        ke = ((_S - 1 - 3 * _CH) // 16) * 16
        le = lane == (_S - 1 - 3 * _CH) % 16
        pos_buf[pl.ds(ke, 16)] = jnp.where(le, zero, pos_buf[pl.ds(ke, 16)])

    cp1 = pltpu.make_async_copy(pos_buf, pos_hbm.at[b, pl.ds(cs, _CH)], sem1)
    cp2 = pltpu.make_async_copy(beh_buf, beh_hbm.at[b, pl.ds(cs, _CH)], sem2)
    cp1.start()
    cp2.start()
    cp1.wait()
    cp2.wait()


@jax.jit
def kernel(input_id_sequence):
    mesh = plsc.VectorSubcoreMesh(
        core_axis_name="c", subcore_axis_name="s", num_cores=1)
    run = pl.kernel(
        _router_body,
        mesh=mesh,
        out_type=(
            jax.ShapeDtypeStruct((_B, _S), jnp.int32),
            jax.ShapeDtypeStruct((_B, _S), jnp.int32),
        ),
        scratch_types=[
            pltpu.VMEM((_BUFSZ,), jnp.int32),
            pltpu.VMEM((_CH,), jnp.int32),
            pltpu.VMEM((_CH,), jnp.int32),
            pltpu.SemaphoreType.DMA,
            pltpu.SemaphoreType.DMA,
        ],
        compiler_params=pltpu.CompilerParams(use_tc_tiling_on_sc=False),
    )
    return run(input_id_sequence)


# speedup vs baseline: 2.0063x; 1.1062x over previous
import jax
import jax.numpy as jnp
from jax import lax
from jax.experimental import pallas as pl
from jax.experimental.pallas import tpu as pltpu
from jax.experimental.pallas import tpu_sc as plsc

_B = 4
_S = 8190
_SP = 8192
_NW = 16
_CW = _SP // _NW
_HALO = 128
_LD = _CW + _HALO
_VPR = _CW // 16
_DOFF = 128


def _router_body(x_hbm, pos_hbm, beh_hbm, buf, pos_buf, beh_buf, sem1, sem2):
    wid = lax.axis_index("s")
    cs = pl.multiple_of(wid * _CW, _HALO)
    base = pl.multiple_of(jnp.maximum(cs - _HALO, 0), _HALO)
    off = cs - base

    pltpu.sync_copy(x_hbm.at[:, pl.ds(base, _LD)],
                    buf.at[:, pl.ds(_DOFF, _B * _LD)].reshape(_B, _LD))

    lane = lax.iota(jnp.int32, 16)
    d = (lane + 3) & 3
    is1 = d == 1
    is2 = d == 2
    d0 = d == 0
    dp1 = d + 1
    zero = jnp.zeros((16,), jnp.int32)
    one_u = jnp.ones((16,), jnp.uint32)

    def do_vec(li0, ko):
        x = buf[0, pl.ds(li0, 16)]
        x1 = buf[0, pl.ds(li0 - 1, 16)]
        x2 = buf[0, pl.ds(li0 - 2, 16)]
        x3 = buf[0, pl.ds(li0 - 3, 16)]
        v = jnp.where(is1, x1, jnp.where(is2, x2, x3))
        pos_kill = plsc.bitcast(x, jnp.uint32) <= one_u
        pos = jnp.where(pos_kill, zero, dp1)
        beh_kill = d0 | (v == 1)
        beh = jnp.where(beh_kill, zero, v)
        pos_buf[0, pl.ds(ko, 16)] = pos
        beh_buf[0, pl.ds(ko, 16)] = beh

    def row(b, _):
        li_row = _DOFF + b * _LD + off

        def body(i, _):
            k = i * 64
            for u in range(4):
                do_vec(li_row + k + u * 16, b * _CW + k + u * 16)
            return 0

        lax.fori_loop(0, _VPR // 4, body, 0)
        return 0

    lax.fori_loop(0, _B, row, 0)

    @pl.when(wid == 0)
    def _():
        l0 = lane == 0
        for b in range(_B):
            o = b * _CW
            pos_buf[0, pl.ds(o, 16)] = jnp.where(
                l0, zero, pos_buf[0, pl.ds(o, 16)])
            beh_buf[0, pl.ds(o, 16)] = jnp.where(
                l0, zero, beh_buf[0, pl.ds(o, 16)])

    @pl.when(wid == _NW - 1)
    def _():
        le = lane == 13
        for b in range(_B):
            o = b * _CW + _CW - 16
            pos_buf[0, pl.ds(o, 16)] = jnp.where(
                le, zero, pos_buf[0, pl.ds(o, 16)])

    cp1 = pltpu.make_async_copy(
        pos_buf.reshape(_B, _CW), pos_hbm.at[:, pl.ds(cs, _CW)], sem1)
    cp2 = pltpu.make_async_copy(
        beh_buf.reshape(_B, _CW), beh_hbm.at[:, pl.ds(cs, _CW)], sem2)
    cp1.start()
    cp2.start()
    cp1.wait()
    cp2.wait()


@jax.jit
def kernel(input_id_sequence):
    mesh = plsc.VectorSubcoreMesh(
        core_axis_name="c", subcore_axis_name="s", num_cores=1)
    run = pl.kernel(
        _router_body,
        mesh=mesh,
        out_type=(
            jax.ShapeDtypeStruct((_B, _S), jnp.int32),
            jax.ShapeDtypeStruct((_B, _S), jnp.int32),
        ),
        scratch_types=[
            pltpu.VMEM((1, _DOFF + _B * _LD), jnp.int32),
            pltpu.VMEM((1, _B * _CW), jnp.int32),
            pltpu.VMEM((1, _B * _CW), jnp.int32),
            pltpu.SemaphoreType.DMA,
            pltpu.SemaphoreType.DMA,
        ],
    )
    return run(input_id_sequence)
